# baseline (speedup 1.0000x reference)
import functools

import jax
import jax.numpy as jnp
from jax import lax
from jax.experimental import pallas as pl
from jax.experimental.pallas import tpu as pltpu
from jax.experimental.pallas import tpu_sc as plsc

VOCAB = 1000000
EMB = 64
EMBP = 128
B = 4096
L = 200
EPS = 1e-12

NC = 2
NS = 16
NW = NC * NS
BBLK = B // NW
NG = BBLK // 16
NEB = EMB // 8


def _rsqrt_newton(x):
    i = lax.bitcast_convert_type(x, jnp.int32)
    i = jnp.full((16,), 0x5F3759DF, dtype=jnp.int32) - lax.shift_right_logical(
        i, jnp.full((16,), 1, dtype=jnp.int32))
    y = lax.bitcast_convert_type(i, jnp.float32)
    half_x = x * 0.5
    for _ in range(3):
        y = y * (1.5 - half_x * y * y)
    return y


def _sc_body(ids_hbm, tbl_hbm, gamma_hbm, beta_hbm, out_hbm,
             g_v, b_v, sc_v, sh_v,
             idx0, idx1, rows0, rows1, tiles0, tiles1,
             si0, si1, sg0, sg1, so0, so1):
    wid = lax.axis_index("s") * NC + lax.axis_index("c")

    pltpu.sync_copy(gamma_hbm, g_v)
    pltpu.sync_copy(beta_hbm, b_v)

    iota16 = lax.iota(jnp.int32, 16)
    zeros16 = iota16 * 0

    def idx_copy(l, idxv, sem):
        return pltpu.make_async_copy(
            ids_hbm.at[pl.ds(l * B + wid * BBLK, BBLK)], idxv, sem)

    def gather_copy(idxv, rowsv, sem):
        return pltpu.make_async_copy(tbl_hbm.at[idxv], rowsv, sem)

    def out_copy(l, tilesv, sem):
        return pltpu.make_async_copy(tilesv, out_hbm.at[l, :, wid, :], sem)

    def compute(rowsv, tilesv):
        rots = [(iota16 + d) & 15 for d in range(16)]

        @plsc.parallel_loop(0, NG)
        def p1_group(g):
            row_idx = g * 16 + iota16
            z = zeros16 * 0.0
            accs = [z] * 4
            sqs = [z] * 4
            for c in range(4):
                base = c * 16
                for d in range(16):
                    v = plsc.load_gather(rowsv, [row_idx, base + rots[d]])
                    accs[c] = accs[c] + v
                    sqs[c] = sqs[c] + v * v
            s = (accs[0] + accs[1]) + (accs[2] + accs[3])
            sq = (sqs[0] + sqs[1]) + (sqs[2] + sqs[3])
            mean = s * (1.0 / EMB)
            var = sq * (1.0 / EMB) - mean * mean
            inv = _rsqrt_newton(var + EPS)
            sc_v[pl.ds(g * 16, 16)] = inv
            sh_v[pl.ds(g * 16, 16)] = -(mean * inv)

        ags = [sc_v[pl.ds(g * 16, 16)] for g in range(NG)]
        bgs = [sh_v[pl.ds(g * 16, 16)] for g in range(NG)]

        @plsc.parallel_loop(0, 16)
        def p2_d(dd):
            rot = (iota16 + dd) & 15
            hi = lax.shift_right_logical(rot, jnp.full((16,), 3, jnp.int32))
            slo = rot & 7
            col0 = slo * 128 + iota16
            for c in range(4):
                col = c * 16 + rot
                row0 = 2 * c + hi
                grot = jnp.take_along_axis(
                    g_v[pl.ds(c * 16, 16)], rot, axis=0)
                brot = jnp.take_along_axis(
                    b_v[pl.ds(c * 16, 16)], rot, axis=0)
                xs = [plsc.load_gather(rowsv, [g * 16 + iota16, col])
                      for g in range(NG)]
                os = [(xs[g] * ags[g] + bgs[g]) * grot + brot
                      for g in range(NG)]
                for g in range(NG):
                    plsc.store_scatter(tilesv, [row0, col0 + g * 16], os[g])

    def round_step(l, bufs):
        idxv, rowsv, tilesv, si, sg, so, idxn, rowsn, sgn, sin = bufs
        gather_copy(idxv, rowsv, sg).wait()

        @pl.when(l + 2 < L)
        def _():
            idx_copy(l + 2, idxv, si).start()

        @pl.when(l + 1 < L)
        def _():
            idx_copy(l + 1, idxn, sin).wait()
            gather_copy(idxn, rowsn, sgn).start()

        @pl.when(l >= 2)
        def _():
            out_copy(l, tilesv, so).wait()

        compute(rowsv, tilesv)
        out_copy(l, tilesv, so).start()

    idx_copy(0, idx0, si0).start()
    idx_copy(1, idx1, si1).start()
    idx_copy(0, idx0, si0).wait()
    gather_copy(idx0, rows0, sg0).start()

    def k_body(k, carry):
        l0 = 2 * k
        round_step(l0, (idx0, rows0, tiles0, si0, sg0, so0,
                        idx1, rows1, sg1, si1))
        round_step(l0 + 1, (idx1, rows1, tiles1, si1, sg1, so1,
                            idx0, rows0, sg0, si0))
        return carry

    lax.fori_loop(0, L // 2, k_body, 0)

    out_copy(L - 2, tiles0, so0).wait()
    out_copy(L - 1, tiles1, so1).wait()


@jax.jit
def _run(ids_lmajor, tbl_padded, gamma, beta):
    mesh = plsc.VectorSubcoreMesh(core_axis_name="c", subcore_axis_name="s")
    k = functools.partial(
        pl.kernel,
        out_type=jax.ShapeDtypeStruct((L, NEB, NW, 8 * 128), jnp.float32),
        mesh=mesh,
        scratch_types=[
            pltpu.VMEM((EMB,), jnp.float32),
            pltpu.VMEM((EMB,), jnp.float32),
            pltpu.VMEM((BBLK,), jnp.float32),
            pltpu.VMEM((BBLK,), jnp.float32),
            pltpu.VMEM((BBLK,), jnp.int32),
            pltpu.VMEM((BBLK,), jnp.int32),
            pltpu.VMEM((BBLK, EMBP), jnp.float32),
            pltpu.VMEM((BBLK, EMBP), jnp.float32),
            pltpu.VMEM((NEB, 8 * 128), jnp.float32),
            pltpu.VMEM((NEB, 8 * 128), jnp.float32),
            pltpu.SemaphoreType.DMA,
            pltpu.SemaphoreType.DMA,
            pltpu.SemaphoreType.DMA,
            pltpu.SemaphoreType.DMA,
            pltpu.SemaphoreType.DMA,
            pltpu.SemaphoreType.DMA,
        ],
        compiler_params=pltpu.CompilerParams(
            use_tc_tiling_on_sc=False, needs_layout_passes=False),
    )(_sc_body)
    return k(ids_lmajor, tbl_padded, gamma, beta)


def kernel(input_ids, table, gamma, beta):
    ids_lmajor = jnp.swapaxes(input_ids, 0, 1).reshape(-1).astype(jnp.int32)
    tbl_padded = jnp.pad(table, ((0, 0), (0, EMBP - EMB)))
    out4 = _run(ids_lmajor, tbl_padded, gamma, beta)
    out = (out4.reshape(L, NEB, NW, 8, 128)
           .transpose(2, 4, 0, 1, 3)
           .reshape(B, L, EMB))
    return out


# speedup vs baseline: 1.1587x; 1.1587x over previous
import functools

import jax
import jax.numpy as jnp
from jax import lax
from jax.experimental import pallas as pl
from jax.experimental.pallas import tpu as pltpu
from jax.experimental.pallas import tpu_sc as plsc

VOCAB = 1000000
EMB = 64
EMBP = 128
B = 4096
L = 200
EPS = 1e-12

NC = 2
NS = 16
NW = NC * NS
BBLK = B // NW
NG = BBLK // 16
NEB = EMB // 8


def _rsqrt_newton(x):
    i = lax.bitcast_convert_type(x, jnp.int32)
    i = jnp.full((16,), 0x5F3759DF, dtype=jnp.int32) - lax.shift_right_logical(
        i, jnp.full((16,), 1, dtype=jnp.int32))
    y = lax.bitcast_convert_type(i, jnp.float32)
    half_x = x * 0.5
    for _ in range(3):
        y = y * (1.5 - half_x * y * y)
    return y


def _sc_body(ids_hbm, tbl_hbm, gamma_hbm, beta_hbm, out_hbm,
             g_v, b_v, sc_v, sh_v,
             idx0, idx1, rows0, rows1, tiles0, tiles1,
             si0, si1, sg0, sg1, so0, so1):
    wid = lax.axis_index("s") * NC + lax.axis_index("c")

    pltpu.sync_copy(gamma_hbm, g_v)
    pltpu.sync_copy(beta_hbm, b_v)

    iota16 = lax.iota(jnp.int32, 16)
    zeros16 = iota16 * 0

    def idx_copy(l, idxv, sem):
        return pltpu.make_async_copy(
            ids_hbm.at[pl.ds(l * B + wid * BBLK, BBLK)], idxv, sem)

    def gather_copy(idxv, rowsv, sem):
        return pltpu.make_async_copy(tbl_hbm.at[idxv], rowsv, sem)

    def out_copy(l, tilesv, sem):
        return pltpu.make_async_copy(tilesv, out_hbm.at[l, :, wid, :], sem)

    def compute(rowsv, tilesv):
        rots = [(iota16 + d) & 15 for d in range(16)]

        @plsc.parallel_loop(0, NG)
        def p1_group(g):
            row_idx = g * 16 + iota16
            z = zeros16 * 0.0
            accs = [z] * 4
            sqs = [z] * 4
            for c in range(4):
                base = c * 16
                for d in range(16):
                    v = plsc.load_gather(rowsv, [row_idx, base + rots[d]])
                    accs[c] = accs[c] + v
                    sqs[c] = sqs[c] + v * v
            s = (accs[0] + accs[1]) + (accs[2] + accs[3])
            sq = (sqs[0] + sqs[1]) + (sqs[2] + sqs[3])
            mean = s * (1.0 / EMB)
            var = sq * (1.0 / EMB) - mean * mean
            inv = _rsqrt_newton(var + EPS)
            sc_v[pl.ds(g * 16, 16)] = inv
            sh_v[pl.ds(g * 16, 16)] = -(mean * inv)

        ags = [sc_v[pl.ds(g * 16, 16)] for g in range(NG)]
        bgs = [sh_v[pl.ds(g * 16, 16)] for g in range(NG)]

        @plsc.parallel_loop(0, 4 * 16)
        def p2_cd(cd):
            c = cd >> 4
            dd = cd & 15
            rot = (iota16 + dd) & 15
            col = c * 16 + rot
            hi = lax.shift_right_logical(rot, jnp.full((16,), 3, jnp.int32))
            slo = rot & 7
            row0 = 2 * c + hi
            col0 = slo * 128 + iota16
            grot = jnp.take_along_axis(g_v[pl.ds(c * 16, 16)], rot, axis=0)
            brot = jnp.take_along_axis(b_v[pl.ds(c * 16, 16)], rot, axis=0)
            xs = [plsc.load_gather(rowsv, [g * 16 + iota16, col])
                  for g in range(NG)]
            os = [(xs[g] * ags[g] + bgs[g]) * grot + brot for g in range(NG)]
            for g in range(NG):
                plsc.store_scatter(tilesv, [row0, col0 + g * 16], os[g])

    def round_step(l, bufs):
        idxv, rowsv, tilesv, si, sg, so, idxn, rowsn, sgn, sin = bufs
        gather_copy(idxv, rowsv, sg).wait()

        @pl.when(l + 2 < L)
        def _():
            idx_copy(l + 2, idxv, si).start()

        @pl.when(l + 1 < L)
        def _():
            idx_copy(l + 1, idxn, sin).wait()
            gather_copy(idxn, rowsn, sgn).start()

        @pl.when(l >= 2)
        def _():
            out_copy(l, tilesv, so).wait()

        compute(rowsv, tilesv)
        out_copy(l, tilesv, so).start()

    idx_copy(0, idx0, si0).start()
    idx_copy(1, idx1, si1).start()
    idx_copy(0, idx0, si0).wait()
    gather_copy(idx0, rows0, sg0).start()

    def k_body(k, carry):
        l0 = 2 * k
        round_step(l0, (idx0, rows0, tiles0, si0, sg0, so0,
                        idx1, rows1, sg1, si1))
        round_step(l0 + 1, (idx1, rows1, tiles1, si1, sg1, so1,
                            idx0, rows0, sg0, si0))
        return carry

    lax.fori_loop(0, L // 2, k_body, 0)

    out_copy(L - 2, tiles0, so0).wait()
    out_copy(L - 1, tiles1, so1).wait()


@jax.jit
def _run(ids_lmajor, tbl_padded, gamma, beta):
    mesh = plsc.VectorSubcoreMesh(core_axis_name="c", subcore_axis_name="s")
    k = functools.partial(
        pl.kernel,
        out_type=jax.ShapeDtypeStruct((L, NEB, NW, 8 * 128), jnp.float32),
        mesh=mesh,
        scratch_types=[
            pltpu.VMEM((EMB,), jnp.float32),
            pltpu.VMEM((EMB,), jnp.float32),
            pltpu.VMEM((BBLK,), jnp.float32),
            pltpu.VMEM((BBLK,), jnp.float32),
            pltpu.VMEM((BBLK,), jnp.int32),
            pltpu.VMEM((BBLK,), jnp.int32),
            pltpu.VMEM((BBLK, EMBP), jnp.float32),
            pltpu.VMEM((BBLK, EMBP), jnp.float32),
            pltpu.VMEM((NEB, 8 * 128), jnp.float32),
            pltpu.VMEM((NEB, 8 * 128), jnp.float32),
            pltpu.SemaphoreType.DMA,
            pltpu.SemaphoreType.DMA,
            pltpu.SemaphoreType.DMA,
            pltpu.SemaphoreType.DMA,
            pltpu.SemaphoreType.DMA,
            pltpu.SemaphoreType.DMA,
        ],
        compiler_params=pltpu.CompilerParams(
            use_tc_tiling_on_sc=False, needs_layout_passes=False),
    )(_sc_body)
    return k(ids_lmajor, tbl_padded, gamma, beta)


def kernel(input_ids, table, gamma, beta):
    ids_lmajor = jnp.swapaxes(input_ids, 0, 1).reshape(-1).astype(jnp.int32)
    tbl_padded = jnp.pad(table, ((0, 0), (0, EMBP - EMB)))
    out4 = _run(ids_lmajor, tbl_padded, gamma, beta)
    out = (out4.reshape(L, NEB, NW, 8, 128)
           .transpose(2, 4, 0, 1, 3)
           .reshape(B, L, EMB))
    return out


# speedup vs baseline: 1.3426x; 1.1587x over previous
import functools

import jax
import jax.numpy as jnp
from jax import lax
from jax.experimental import pallas as pl
from jax.experimental.pallas import tpu as pltpu
from jax.experimental.pallas import tpu_sc as plsc

VOCAB = 1000000
EMB = 64
EMBP = 128
B = 4096
L = 200
EPS = 1e-12

NC = 2
NS = 16
NW = NC * NS
BBLK = B // NW
NG = BBLK // 16
NEB = EMB // 8


def _rsqrt_newton(x):
    i = lax.bitcast_convert_type(x, jnp.int32)
    i = jnp.full((16,), 0x5F3759DF, dtype=jnp.int32) - lax.shift_right_logical(
        i, jnp.full((16,), 1, dtype=jnp.int32))
    y = lax.bitcast_convert_type(i, jnp.float32)
    half_x = x * 0.5
    for _ in range(3):
        y = y * (1.5 - half_x * y * y)
    return y


def _sc_body(ids_hbm, tbl_hbm, gamma_hbm, beta_hbm, out_hbm,
             g_v, b_v, sc_v, sh_v, ps_v, qs_v,
             idx0, idx1, rows0, rows1, tiles0, tiles1,
             si0, si1, sg0, sg1, so0, so1):
    wid = lax.axis_index("s") * NC + lax.axis_index("c")

    pltpu.sync_copy(gamma_hbm, g_v)
    pltpu.sync_copy(beta_hbm, b_v)

    iota16 = lax.iota(jnp.int32, 16)
    zeros16 = iota16 * 0

    def idx_copy(l, idxv, sem):
        return pltpu.make_async_copy(
            ids_hbm.at[pl.ds(l * B + wid * BBLK, BBLK)], idxv, sem)

    def gather_copy(idxv, rowsv, sem):
        return pltpu.make_async_copy(tbl_hbm.at[idxv], rowsv, sem)

    def out_copy(l, tilesv, sem):
        return pltpu.make_async_copy(tilesv, out_hbm.at[l, :, wid, :], sem)

    def compute(rowsv, tilesv):
        rots = [(iota16 + d) & 15 for d in range(16)]

        @plsc.parallel_loop(0, NG * 4, unroll=2)
        def p1_part(gc):
            g = gc >> 2
            c = gc & 3
            row_idx = g * 16 + iota16
            base = c * 16
            z = zeros16 * 0.0
            a0 = a1 = q0 = q1 = z
            for d in range(0, 16, 2):
                v0 = plsc.load_gather(rowsv, [row_idx, base + rots[d]])
                v1 = plsc.load_gather(rowsv, [row_idx, base + rots[d + 1]])
                a0 = a0 + v0
                a1 = a1 + v1
                q0 = q0 + v0 * v0
                q1 = q1 + v1 * v1
            ps_v[g, pl.ds(c * 16, 16)] = a0 + a1
            qs_v[g, pl.ds(c * 16, 16)] = q0 + q1

        @plsc.parallel_loop(0, NG)
        def p1_group(g):
            s = ((ps_v[g, pl.ds(0, 16)] + ps_v[g, pl.ds(16, 16)])
                 + (ps_v[g, pl.ds(32, 16)] + ps_v[g, pl.ds(48, 16)]))
            sq = ((qs_v[g, pl.ds(0, 16)] + qs_v[g, pl.ds(16, 16)])
                  + (qs_v[g, pl.ds(32, 16)] + qs_v[g, pl.ds(48, 16)]))
            mean = s * (1.0 / EMB)
            var = sq * (1.0 / EMB) - mean * mean
            inv = _rsqrt_newton(var + EPS)
            sc_v[pl.ds(g * 16, 16)] = inv
            sh_v[pl.ds(g * 16, 16)] = -(mean * inv)

        ags = [sc_v[pl.ds(g * 16, 16)] for g in range(NG)]
        bgs = [sh_v[pl.ds(g * 16, 16)] for g in range(NG)]

        @plsc.parallel_loop(0, 4 * 16, unroll=2)
        def p2_cd(cd):
            c = cd >> 4
            dd = cd & 15
            rot = (iota16 + dd) & 15
            col = c * 16 + rot
            hi = lax.shift_right_logical(rot, jnp.full((16,), 3, jnp.int32))
            slo = rot & 7
            row0 = 2 * c + hi
            col0 = slo * 128 + iota16
            grot = jnp.take_along_axis(g_v[pl.ds(c * 16, 16)], rot, axis=0)
            brot = jnp.take_along_axis(b_v[pl.ds(c * 16, 16)], rot, axis=0)
            xs = [plsc.load_gather(rowsv, [g * 16 + iota16, col])
                  for g in range(NG)]
            os = [(xs[g] * ags[g] + bgs[g]) * grot + brot for g in range(NG)]
            for g in range(NG):
                plsc.store_scatter(tilesv, [row0, col0 + g * 16], os[g])

    def round_step(l, bufs):
        idxv, rowsv, tilesv, si, sg, so, idxn, rowsn, sgn, sin = bufs
        gather_copy(idxv, rowsv, sg).wait()

        @pl.when(l + 2 < L)
        def _():
            idx_copy(l + 2, idxv, si).start()

        @pl.when(l + 1 < L)
        def _():
            idx_copy(l + 1, idxn, sin).wait()
            gather_copy(idxn, rowsn, sgn).start()

        @pl.when(l >= 2)
        def _():
            out_copy(l, tilesv, so).wait()

        compute(rowsv, tilesv)
        out_copy(l, tilesv, so).start()

    idx_copy(0, idx0, si0).start()
    idx_copy(1, idx1, si1).start()
    idx_copy(0, idx0, si0).wait()
    gather_copy(idx0, rows0, sg0).start()

    def k_body(k, carry):
        l0 = 2 * k
        round_step(l0, (idx0, rows0, tiles0, si0, sg0, so0,
                        idx1, rows1, sg1, si1))
        round_step(l0 + 1, (idx1, rows1, tiles1, si1, sg1, so1,
                            idx0, rows0, sg0, si0))
        return carry

    lax.fori_loop(0, L // 2, k_body, 0)

    out_copy(L - 2, tiles0, so0).wait()
    out_copy(L - 1, tiles1, so1).wait()


@jax.jit
def _run(ids_lmajor, tbl_padded, gamma, beta):
    mesh = plsc.VectorSubcoreMesh(core_axis_name="c", subcore_axis_name="s")
    k = functools.partial(
        pl.kernel,
        out_type=jax.ShapeDtypeStruct((L, NEB, NW, 8 * 128), jnp.float32),
        mesh=mesh,
        scratch_types=[
            pltpu.VMEM((EMB,), jnp.float32),
            pltpu.VMEM((EMB,), jnp.float32),
            pltpu.VMEM((BBLK,), jnp.float32),
            pltpu.VMEM((BBLK,), jnp.float32),
            pltpu.VMEM((NG, EMB), jnp.float32),
            pltpu.VMEM((NG, EMB), jnp.float32),
            pltpu.VMEM((BBLK,), jnp.int32),
            pltpu.VMEM((BBLK,), jnp.int32),
            pltpu.VMEM((BBLK, EMBP), jnp.float32),
            pltpu.VMEM((BBLK, EMBP), jnp.float32),
            pltpu.VMEM((NEB, 8 * 128), jnp.float32),
            pltpu.VMEM((NEB, 8 * 128), jnp.float32),
            pltpu.SemaphoreType.DMA,
            pltpu.SemaphoreType.DMA,
            pltpu.SemaphoreType.DMA,
            pltpu.SemaphoreType.DMA,
            pltpu.SemaphoreType.DMA,
            pltpu.SemaphoreType.DMA,
        ],
        compiler_params=pltpu.CompilerParams(
            use_tc_tiling_on_sc=False, needs_layout_passes=False),
    )(_sc_body)
    return k(ids_lmajor, tbl_padded, gamma, beta)


def kernel(input_ids, table, gamma, beta):
    ids_lmajor = jnp.swapaxes(input_ids, 0, 1).reshape(-1).astype(jnp.int32)
    tbl_padded = jnp.pad(table, ((0, 0), (0, EMBP - EMB)))
    out4 = _run(ids_lmajor, tbl_padded, gamma, beta)
    out = (out4.reshape(L, NEB, NW, 8, 128)
           .transpose(2, 4, 0, 1, 3)
           .reshape(B, L, EMB))
    return out
